# baseline (speedup 1.0000x reference)
import functools

import jax
import jax.numpy as jnp
import numpy as np
from jax import lax
from jax.experimental import pallas as pl
from jax.experimental.pallas import tpu as pltpu
from jax.experimental.pallas import tpu_sc as plsc

_NS = [65536, 16384, 4096, 1024, 256, 64]
_EPS = 1e-5

_SC_CORES = 2
_SC_SUBCORES = 16
_SPMEM_BUDGET = 6_500_000


def _seg_sum_sc(x, src, dst, sel, itp, n):
    C = x.shape[1]
    E = src.shape[0]
    rows_total = 9 * n
    half = rows_total // _SC_CORES
    smax = max(16, (_SPMEM_BUDGET // (C * 4)) // 16 * 16)
    npass = -(-half // smax)
    slab = -(-half // npass // 16) * 16
    slabs = []
    off = 0
    while off < half:
        s = min(slab, half - off)
        slabs.append((off, s))
        off += s

    Es = E // _SC_SUBCORES
    B = min(512, Es)
    nb = Es // B
    G = min(64, B)
    HB = B + G + 16
    Z = min(512, 65536 // C)
    acc_rows = max(s for _, s in slabs)

    mesh = plsc.VectorSubcoreMesh(core_axis_name="c", subcore_axis_name="s",
                                  num_cores=_SC_CORES,
                                  num_subcores=_SC_SUBCORES)

    @functools.partial(
        pl.kernel, mesh=mesh,
        compiler_params=pltpu.CompilerParams(use_tc_tiling_on_sc=False,
                                             needs_layout_passes=False),
        out_type=jax.ShapeDtypeStruct((rows_total, C), jnp.float32),
        scratch_types=[
            pltpu.VMEM((B,), jnp.int32),
            pltpu.VMEM((B,), jnp.int32),
            pltpu.VMEM((B,), jnp.int32),
            pltpu.VMEM((B,), jnp.float32),
            pltpu.VMEM((HB,), jnp.int32),
            pltpu.VMEM((HB,), jnp.int32),
            pltpu.VMEM((HB,), jnp.float32),
            pltpu.VMEM((G,), jnp.int32),
            pltpu.VMEM((G,), jnp.int32),
            pltpu.VMEM((G, C), jnp.float32),
            pltpu.VMEM((G, C), jnp.float32),
            pltpu.VMEM((Z, C), jnp.float32),
            pltpu.VMEM_SHARED((acc_rows, C), jnp.float32),
            pltpu.SemaphoreType.DMA,
        ])
    def k(x_h, src_h, dst_h, sel_h, itp_h, agg_h,
          src_v, dst_v, sel_v, itp_v, hg, hs, hp,
          grow0, grow1, rows0, rows1, zbuf, acc, sem):
        cid = lax.axis_index("c")
        sid = lax.axis_index("s")
        zero16 = jnp.zeros((16,), jnp.float32)

        def _zb(r, _):
            for cc in range(C // 16):
                zbuf[r, pl.ds(cc * 16, 16)] = zero16
            return 0
        lax.fori_loop(0, Z, _zb, 0)

        for off, s in slabs:
            glo = cid * half + off
            rps = s // _SC_SUBCORES
            z0 = sid * rps
            for kk in range(0, rps, Z):
                w = min(Z, rps - kk)
                pltpu.sync_copy(zbuf.at[pl.ds(0, w)],
                                acc.at[pl.ds(z0 + kk, w)])
            plsc.subcore_barrier()

            def batch(b, _):
                e0 = sid * Es + b * B
                pltpu.sync_copy(src_h.at[pl.ds(e0, B)], src_v)
                pltpu.sync_copy(dst_h.at[pl.ds(e0, B)], dst_v)
                pltpu.sync_copy(sel_h.at[pl.ds(e0, B)], sel_v)
                pltpu.sync_copy(itp_h.at[pl.ds(e0, B)], itp_v)

                def _pf(q, _):
                    o = q * 16
                    hg[pl.ds(o, 16)] = jnp.zeros((16,), jnp.int32)
                    hs[pl.ds(o, 16)] = jnp.zeros((16,), jnp.int32)
                    hp[pl.ds(o, 16)] = zero16
                    return 0
                lax.fori_loop(0, HB // 16, _pf, 0)

                def _cmp(q, kacc):
                    o = q * 16
                    g = (sel_v[pl.ds(o, 16)] * n + dst_v[pl.ds(o, 16)]
                         - glo)
                    m = (g >= 0) & (g < s)
                    mi = m.astype(jnp.int32)
                    inc = plsc.cumsum(mi)
                    pos = kacc + inc - mi
                    plsc.store_scatter(hg, [pos], g, mask=m)
                    plsc.store_scatter(hs, [pos], src_v[pl.ds(o, 16)],
                                       mask=m)
                    plsc.store_scatter(hp, [pos],
                                       itp_v[pl.ds(o, 16)].astype(
                                           jnp.float32), mask=m)
                    return kacc + jnp.max(inc, axis=0)
                kh = lax.fori_loop(0, B // 16, _cmp, 0)

                def drain(jb, _):
                    j0 = jb * G
                    grow, rows = grow0, rows0
                    for t in range(G // 16):
                        grow[pl.ds(t * 16, 16)] = hg[pl.ds(j0 + t * 16, 16)]
                    pltpu.async_copy(x_h.at[hs.at[pl.ds(j0, G)]],
                                     rows, sem).wait()

                    def _scale(gq, _):
                        o2 = gq * 16
                        t16 = hp[pl.ds(j0 + o2, 16)]
                        for i in range(16):
                            bc = t16.at[jnp.full((16,), i, jnp.int32)
                                        ].get(mode='promise_in_bounds')
                            for cc in range(C // 16):
                                rows[o2 + i, pl.ds(cc * 16, 16)] = (
                                    rows[o2 + i, pl.ds(cc * 16, 16)] * bc)
                        return 0
                    lax.fori_loop(0, G // 16, _scale, 0)
                    pltpu.sync_copy(rows, acc.at[grow], add=True)
                    return 0
                lax.fori_loop(0, (kh + G - 1) // G, drain, 0)
                plsc.subcore_barrier()
                return 0
            lax.fori_loop(0, nb, batch, 0)

            for kk in range(0, rps, Z):
                w = min(Z, rps - kk)
                pltpu.sync_copy(acc.at[pl.ds(z0 + kk, w)],
                                agg_h.at[pl.ds(glo + z0 + kk, w)])
            plsc.subcore_barrier()

    return k(x, src, dst, sel, itp)


def _bn_apply(y, g, be, relu):
    m = jnp.mean(y, axis=0)
    v = jnp.var(y, axis=0)
    out = (y - m) / jnp.sqrt(v + _EPS) * g + be
    return jax.nn.relu(out) if relu else out


def _pool_max(x, cluster, n_out):
    out = jax.ops.segment_max(x, cluster, num_segments=n_out)
    return jnp.where(jnp.isfinite(out), out, 0.0)


def _row_gather(T, idx):
    return T[idx]


def _sel_conv(x, ei, sel, itp, W9, b, n):
    ci = x.shape[1]
    cp = -ci % 16
    if cp:
        x = jnp.pad(x, ((0, 0), (0, cp)))
        W9 = jnp.pad(W9, ((0, 0), (0, cp), (0, 0)))
    agg = _seg_sum_sc(x, ei[0], ei[1], sel, itp, n)
    agg = agg.reshape(9, n, ci + cp)
    if cp:
        agg = agg[:, :, :ci]
        W9 = W9[:, :ci, :]
    out = jnp.einsum('sni,sio->no', agg, W9)
    return out if b is None else out + b


def _res_block(x_parts, ei, sel, itp, p, n):
    x = (x_parts[0] if len(x_parts) == 1
         else jnp.concatenate(x_parts, axis=1))
    h1 = x @ p['W1'] + p['b1']
    h1n = _bn_apply(h1, p['g1'], p['be1'], relu=True)
    cv = _sel_conv(h1n, ei, sel, itp, p['W2'], p['b2'], n)
    h2n = _bn_apply(cv, p['g2'], p['be2'], relu=True)
    out3 = h2n @ p['W3'] + p['b3']
    xr = x @ p['Wr'] + p['br']
    res = _bn_apply(xr, p['gr'], p['ber'], relu=False)
    return jax.nn.relu(out3 + res)


def _forward(x, eis, sels, itps, clus, params):
    ns = _NS
    enc1 = _sel_conv(x, eis[0], sels[0], itps[0], params['start_W'],
                     params['start_b'], ns[0])
    enc = [enc1]
    h = enc1
    for l in range(1, 6):
        pooled = _pool_max(h, clus[l - 1], ns[l])
        h = _res_block([pooled], eis[l], sels[l], itps[l],
                       params['enc%d' % l], ns[l])
        enc.append(h)
    dec = enc[5]
    for l in range(4, -1, -1):
        up = _row_gather(dec, clus[l])
        dec = _res_block([up, enc[l]], eis[l], sels[l], itps[l],
                         params['dec%d' % (l + 1)], ns[l])
    return _sel_conv(dec, eis[0], sels[0], itps[0], params['final_W'],
                     params['final_b'], ns[0])


def kernel(x, edge_index_0, edge_index_1, edge_index_2, edge_index_3,
           edge_index_4, edge_index_5, selections_0, selections_1,
           selections_2, selections_3, selections_4, selections_5,
           interps_0, interps_1, interps_2, interps_3, interps_4, interps_5,
           cluster_0, cluster_1, cluster_2, cluster_3, cluster_4, params):
    eis = [edge_index_0, edge_index_1, edge_index_2, edge_index_3,
           edge_index_4, edge_index_5]
    sels = [selections_0, selections_1, selections_2, selections_3,
            selections_4, selections_5]
    itps = [interps_0, interps_1, interps_2, interps_3, interps_4, interps_5]
    clus = [cluster_0, cluster_1, cluster_2, cluster_3, cluster_4]
    return _forward(x, eis, sels, itps, clus, params)
